# Initial kernel scaffold; baseline (speedup 1.0000x reference)
import jax
import jax.numpy as jnp
from jax.experimental import pallas as pl


def kernel(tables, indices, seq_lens):
    raise NotImplementedError("write your pallas kernel here")


# speedup vs baseline: 6.3900x; 6.3900x over previous
import functools

import jax
import jax.numpy as jnp
from jax import lax
from jax.experimental import pallas as pl
from jax.experimental.pallas import tpu as pltpu
from jax.experimental.pallas import tpu_sc as plsc

F = 26
VOCAB = 100000
EMB = 32
B = 1024
T = 200

N = B * T * F
NC = 2
NS = 16
NW = NC * NS
R = N // NW
GATHER = 128
K = 10
CHUNK = K * GATHER
NCHUNK = R // CHUNK
ROWS_PER_IDXROW = R // GATHER

_mesh = plsc.VectorSubcoreMesh(
    core_axis_name="c", subcore_axis_name="s", num_cores=NC, num_subcores=NS
)


@functools.partial(
    pl.kernel,
    mesh=_mesh,
    out_type=jax.ShapeDtypeStruct((N, EMB), jnp.float32),
    compiler_params=pltpu.CompilerParams(use_tc_tiling_on_sc=False),
    scratch_types=[
        pltpu.VMEM((CHUNK,), jnp.int32),
        pltpu.VMEM((CHUNK, EMB), jnp.float32),
        pltpu.SemaphoreType.DMA,
    ],
)
def _gather_kernel(table_hbm, gidx_hbm, out_hbm, idx_v, rows_v, sem):
    wid = lax.axis_index("s") * NC + lax.axis_index("c")

    def chunk_body(g, carry):
        row0 = wid * R + g * CHUNK
        pltpu.sync_copy(gidx_hbm.at[pl.ds(row0, CHUNK)], idx_v)
        copies = []
        for j in range(K):
            copies.append(
                pltpu.async_copy(
                    table_hbm.at[idx_v.at[pl.ds(j * GATHER, GATHER)]],
                    rows_v.at[pl.ds(j * GATHER, GATHER)],
                    sem,
                )
            )
        for c in copies:
            c.wait()
        pltpu.sync_copy(rows_v, out_hbm.at[pl.ds(row0, CHUNK)])
        return carry

    lax.fori_loop(0, NCHUNK, chunk_body, 0)


def kernel(tables, indices, seq_lens):
    table_flat = tables.reshape(F * VOCAB, EMB)
    offs = jnp.arange(F, dtype=jnp.int32) * VOCAB
    gidx = (indices + offs[None, None, :]).reshape(N)
    out_flat = _gather_kernel(table_flat, gidx)
    return out_flat.reshape(B, T, F * EMB)


# speedup vs baseline: 6.5544x; 1.0257x over previous
import functools

import jax
import jax.numpy as jnp
from jax import lax
from jax.experimental import pallas as pl
from jax.experimental.pallas import tpu as pltpu
from jax.experimental.pallas import tpu_sc as plsc

F = 26
VOCAB = 100000
EMB = 32
B = 1024
T = 200

BT = B * T
NC = 2
NS = 16
NW = NC * NS
ROWS_W = BT // NW
Q = 64
NCHUNK = ROWS_W // Q

_mesh = plsc.VectorSubcoreMesh(
    core_axis_name="c", subcore_axis_name="s", num_cores=NC, num_subcores=NS
)


@functools.partial(
    pl.kernel,
    mesh=_mesh,
    out_type=jax.ShapeDtypeStruct((BT, F * EMB), jnp.float32),
    compiler_params=pltpu.CompilerParams(use_tc_tiling_on_sc=False),
    scratch_types=[
        pltpu.VMEM((F, Q), jnp.int32),
        pltpu.VMEM((F * Q, EMB), jnp.float32),
        pltpu.SemaphoreType.DMA,
        pltpu.SemaphoreType.DMA,
    ],
)
def _gather_kernel(table_hbm, gidx_hbm, out_hbm, idx_v, stages, semg, semw):
    wid = lax.axis_index("s") * NC + lax.axis_index("c")

    def chunk_body(g, carry):
        bt0 = wid * ROWS_W + g * Q
        pltpu.sync_copy(gidx_hbm.at[:, pl.ds(bt0, Q)], idx_v)

        def fire_gather(f, c):
            pltpu.async_copy(
                table_hbm.at[idx_v.at[f]],
                stages.at[pl.ds(f * Q, Q)],
                semg,
            )
            return c

        lax.fori_loop(0, F, fire_gather, 0)
        pltpu.make_async_copy(table_hbm.at[pl.ds(0, F * Q)], stages, semg).wait()

        def fire_write(f, c):
            pltpu.async_copy(
                stages.at[pl.ds(f * Q, Q)],
                out_hbm.at[pl.ds(bt0, Q), pl.ds(f * EMB, EMB)],
                semw,
            )
            return c

        lax.fori_loop(0, F, fire_write, 0)
        pltpu.make_async_copy(stages, out_hbm.at[pl.ds(bt0, Q)], semw).wait()
        return carry

    lax.fori_loop(0, NCHUNK, chunk_body, 0)


def kernel(tables, indices, seq_lens):
    table_flat = tables.reshape(F * VOCAB, EMB)
    offs = jnp.arange(F, dtype=jnp.int32) * VOCAB
    gidx_t = (indices + offs[None, None, :]).transpose(2, 0, 1).reshape(F, BT)
    out2 = _gather_kernel(table_flat, gidx_t)
    return out2.reshape(B, T, F * EMB)
